# baseline (speedup 1.0000x reference)
import jax
import jax.numpy as jnp
from jax import lax
from jax.experimental import pallas as pl
from jax.experimental.pallas import tpu as pltpu
from jax.experimental.pallas import tpu_sc as plsc

N = 6400000
S = 4096
NCORES = 2
NSUB = 16
NW = NCORES * NSUB
P = N // NW
C = 8000
NCHUNK = P // C
G = C // 16
L = 16
BSTRIDE = C + L
W = 3 * S // NSUB


def _splat_last(v):
    idx = jnp.full((L, 1), L - 1, jnp.int32)
    dn = lax.GatherDimensionNumbers(
        offset_dims=(), collapsed_slice_dims=(0,), start_index_map=(0,))
    return lax.gather(v, idx, dn, (1,),
                      mode=lax.GatherScatterMode.PROMISE_IN_BOUNDS)


def _sc_body(x_hbm, y_hbm, z_hbm, q_hbm, b_hbm, out_hbm,
             x_buf, y_buf, z_buf, q_buf, b_buf, acc_x, acc_y, acc_z,
             shared, red, out_loc,
             sem_x0, sem_x1, sem_y0, sem_y1, sem_z0, sem_z1,
             sem_q0, sem_q1, sem_b0, sem_b1):
    cid = lax.axis_index("c")
    sid = lax.axis_index("s")
    wid = cid * NSUB + sid
    base = wid * P

    lane = lax.iota(jnp.int32, L)
    zeros_i = jnp.zeros((L,), jnp.int32)
    zeros_f = jnp.zeros((L,), jnp.float32)
    mask15 = lane == (L - 1)

    sems = ((sem_x0, sem_y0, sem_z0, sem_q0, sem_b0),
            (sem_x1, sem_y1, sem_z1, sem_q1, sem_b1))

    def chunk_copies(c):
        slot = c % 2
        off = base + c * C
        sx, sy, sz, sq, sb = sems[slot]
        return (
            pltpu.make_async_copy(x_hbm.at[pl.ds(off, C)],
                                  x_buf.at[pl.ds(slot * C, C)], sx),
            pltpu.make_async_copy(y_hbm.at[pl.ds(off, C)],
                                  y_buf.at[pl.ds(slot * C, C)], sy),
            pltpu.make_async_copy(z_hbm.at[pl.ds(off, C)],
                                  z_buf.at[pl.ds(slot * C, C)], sz),
            pltpu.make_async_copy(q_hbm.at[pl.ds(off, C)],
                                  q_buf.at[pl.ds(slot * C, C)], sq),
            pltpu.make_async_copy(b_hbm.at[pl.ds(off, C)],
                                  b_buf.at[pl.ds(slot * BSTRIDE + L, C)], sb),
        )

    def zero_body(j, _):
        acc_x[pl.ds(j * L, L)] = zeros_f
        acc_y[pl.ds(j * L, L)] = zeros_f
        acc_z[pl.ds(j * L, L)] = zeros_f
        return 0
    lax.fori_loop(0, S // L, zero_body, 0)

    b_buf[pl.ds(0, L)] = zeros_i

    for cp in chunk_copies(0):
        cp.start()

    carry = (zeros_f, zeros_f, zeros_f)

    for c in range(NCHUNK):
        slot = c % 2
        boff = slot * BSTRIDE
        coff = slot * C
        for cp in chunk_copies(c):
            cp.wait()

        if c + 1 < NCHUNK:
            b_buf[pl.ds((1 - slot) * BSTRIDE, L)] = b_buf[pl.ds(boff + C, L)]
            for cp in chunk_copies(c + 1):
                cp.start()

        def one_group(a, carry):
            cxs, cys, czs = carry
            b = b_buf[pl.ds(boff + L + a, L)]
            bp = b_buf[pl.ds(boff + L - 1 + a, L)]
            qv = q_buf[pl.ds(coff + a, L)]
            dx = x_buf[pl.ds(coff + a, L)] * qv
            dy = y_buf[pl.ds(coff + a, L)] * qv
            dz = z_buf[pl.ds(coff + a, L)] * qv
            lx = plsc.cumsum(dx)
            ly = plsc.cumsum(dy)
            lz = plsc.cumsum(dz)
            ex = (lx - dx) + cxs
            ey = (ly - dy) + cys
            ez = (lz - dz) + czs
            m = b != bp
            plsc.addupdate_scatter(acc_x, [bp], ex, mask=m)
            plsc.addupdate_scatter(acc_y, [bp], ey, mask=m)
            plsc.addupdate_scatter(acc_z, [bp], ez, mask=m)
            plsc.addupdate_scatter(acc_x, [b], -ex, mask=m)
            plsc.addupdate_scatter(acc_y, [b], -ey, mask=m)
            plsc.addupdate_scatter(acc_z, [b], -ez, mask=m)
            return (cxs + _splat_last(lx), cys + _splat_last(ly),
                    czs + _splat_last(lz))

        def group_body(j, carry):
            a = j * (10 * L)
            for u in range(10):
                carry = one_group(a + u * L, carry)
            return carry

        carry = lax.fori_loop(0, G // 10, group_body, carry)

    last_boff = ((NCHUNK - 1) % 2) * BSTRIDE
    vlast = b_buf[pl.ds(last_boff + C, L)]
    cxs, cys, czs = carry
    plsc.addupdate_scatter(acc_x, [vlast], cxs, mask=mask15)
    plsc.addupdate_scatter(acc_y, [vlast], cys, mask=mask15)
    plsc.addupdate_scatter(acc_z, [vlast], czs, mask=mask15)

    row = sid * (3 * S)
    pltpu.sync_copy(acc_x, shared.at[pl.ds(row, S)])
    pltpu.sync_copy(acc_y, shared.at[pl.ds(row + S, S)])
    pltpu.sync_copy(acc_z, shared.at[pl.ds(row + 2 * S, S)])
    plsc.subcore_barrier()

    for t in range(NSUB):
        pltpu.sync_copy(shared.at[pl.ds(t * 3 * S + sid * W, W)],
                        red.at[pl.ds(t * W, W)])

    def red_body(j, _):
        v = red[pl.ds(j * L, L)]
        for t in range(1, NSUB):
            v = v + red[pl.ds(t * W + j * L, L)]
        out_loc[pl.ds(j * L, L)] = v
        return 0
    lax.fori_loop(0, W // L, red_body, 0)

    pltpu.sync_copy(out_loc, out_hbm.at[pl.ds(cid * 3 * S + sid * W, W)])


def _tc_combine(x_ref, o_ref):
    o_ref[...] = x_ref[0] + x_ref[1]


def kernel(positions, q, batch):
    xp = positions[:, 0]
    yp = positions[:, 1]
    zp = positions[:, 2]
    qf = q.reshape(-1)
    b = batch.astype(jnp.int32)

    mesh = plsc.VectorSubcoreMesh(core_axis_name="c", subcore_axis_name="s",
                                  num_cores=NCORES, num_subcores=NSUB)
    sc = pl.kernel(
        _sc_body,
        out_type=jax.ShapeDtypeStruct((NCORES * 3 * S,), jnp.float32),
        mesh=mesh,
        compiler_params=pltpu.CompilerParams(needs_layout_passes=False),
        scratch_types=[
            pltpu.VMEM((2 * C,), jnp.float32),
            pltpu.VMEM((2 * C,), jnp.float32),
            pltpu.VMEM((2 * C,), jnp.float32),
            pltpu.VMEM((2 * C,), jnp.float32),
            pltpu.VMEM((2 * BSTRIDE,), jnp.int32),
            pltpu.VMEM((S,), jnp.float32),
            pltpu.VMEM((S,), jnp.float32),
            pltpu.VMEM((S,), jnp.float32),
            pltpu.VMEM_SHARED((NSUB * 3 * S,), jnp.float32),
            pltpu.VMEM((NSUB * W,), jnp.float32),
            pltpu.VMEM((W,), jnp.float32),
            pltpu.SemaphoreType.DMA,
            pltpu.SemaphoreType.DMA,
            pltpu.SemaphoreType.DMA,
            pltpu.SemaphoreType.DMA,
            pltpu.SemaphoreType.DMA,
            pltpu.SemaphoreType.DMA,
            pltpu.SemaphoreType.DMA,
            pltpu.SemaphoreType.DMA,
            pltpu.SemaphoreType.DMA,
            pltpu.SemaphoreType.DMA,
        ],
    )
    partials = sc(xp, yp, zp, qf, b)

    planar = pl.pallas_call(
        _tc_combine,
        out_shape=jax.ShapeDtypeStruct((3, S), jnp.float32),
    )(partials.reshape(NCORES, 3, S))
    return planar.T


# speedup vs baseline: 1.0103x; 1.0103x over previous
import jax
import jax.numpy as jnp
from jax import lax
from jax.experimental import pallas as pl
from jax.experimental.pallas import tpu as pltpu
from jax.experimental.pallas import tpu_sc as plsc

N = 6400000
S = 4096
NCORES = 2
NSUB = 16
NW = NCORES * NSUB
P = N // NW
C = 8000
NCHUNK = P // C
G = C // 16
L = 16
BSTRIDE = C + L
W = 3 * S // NSUB


def _splat_last(v):
    idx = jnp.full((L, 1), L - 1, jnp.int32)
    dn = lax.GatherDimensionNumbers(
        offset_dims=(), collapsed_slice_dims=(0,), start_index_map=(0,))
    return lax.gather(v, idx, dn, (1,),
                      mode=lax.GatherScatterMode.PROMISE_IN_BOUNDS)


def _sc_body(x_hbm, y_hbm, z_hbm, q_hbm, b_hbm, out_hbm,
             x_buf, y_buf, z_buf, q_buf, b_buf, acc_x, acc_y, acc_z,
             shared, red, out_loc,
             sem_x0, sem_x1, sem_y0, sem_y1, sem_z0, sem_z1,
             sem_q0, sem_q1, sem_b0, sem_b1):
    cid = lax.axis_index("c")
    sid = lax.axis_index("s")
    wid = cid * NSUB + sid
    base = wid * P

    lane = lax.iota(jnp.int32, L)
    zeros_i = jnp.zeros((L,), jnp.int32)
    zeros_f = jnp.zeros((L,), jnp.float32)
    mask15 = lane == (L - 1)

    sems = ((sem_x0, sem_y0, sem_z0, sem_q0, sem_b0),
            (sem_x1, sem_y1, sem_z1, sem_q1, sem_b1))

    def chunk_copies(c):
        slot = c % 2
        off = base + c * C
        sx, sy, sz, sq, sb = sems[slot]
        return (
            pltpu.make_async_copy(x_hbm.at[pl.ds(off, C)],
                                  x_buf.at[pl.ds(slot * C, C)], sx),
            pltpu.make_async_copy(y_hbm.at[pl.ds(off, C)],
                                  y_buf.at[pl.ds(slot * C, C)], sy),
            pltpu.make_async_copy(z_hbm.at[pl.ds(off, C)],
                                  z_buf.at[pl.ds(slot * C, C)], sz),
            pltpu.make_async_copy(q_hbm.at[pl.ds(off, C)],
                                  q_buf.at[pl.ds(slot * C, C)], sq),
            pltpu.make_async_copy(b_hbm.at[pl.ds(off, C)],
                                  b_buf.at[pl.ds(slot * BSTRIDE + L, C)], sb),
        )

    def zero_body(j, _):
        acc_x[pl.ds(j * L, L)] = zeros_f
        acc_y[pl.ds(j * L, L)] = zeros_f
        acc_z[pl.ds(j * L, L)] = zeros_f
        return 0
    lax.fori_loop(0, S // L, zero_body, 0)

    b_buf[pl.ds(0, L)] = zeros_i

    for cp in chunk_copies(0):
        cp.start()

    carry = (zeros_f, zeros_f, zeros_f)

    for c in range(NCHUNK):
        slot = c % 2
        boff = slot * BSTRIDE
        coff = slot * C
        for cp in chunk_copies(c):
            cp.wait()

        if c + 1 < NCHUNK:
            b_buf[pl.ds((1 - slot) * BSTRIDE, L)] = b_buf[pl.ds(boff + C, L)]
            for cp in chunk_copies(c + 1):
                cp.start()

        def one_group(a, carry):
            cxs, cys, czs = carry
            b = b_buf[pl.ds(boff + L + a, L)]
            bp = b_buf[pl.ds(boff + L - 1 + a, L)]
            qv = q_buf[pl.ds(coff + a, L)]
            dx = x_buf[pl.ds(coff + a, L)] * qv
            dy = y_buf[pl.ds(coff + a, L)] * qv
            dz = z_buf[pl.ds(coff + a, L)] * qv
            lx = plsc.cumsum(dx)
            ly = plsc.cumsum(dy)
            lz = plsc.cumsum(dz)
            ex = (lx - dx) + cxs
            ey = (ly - dy) + cys
            ez = (lz - dz) + czs
            m = b != bp
            plsc.addupdate_scatter(acc_x, [bp], ex, mask=m)
            plsc.addupdate_scatter(acc_y, [bp], ey, mask=m)
            plsc.addupdate_scatter(acc_z, [bp], ez, mask=m)
            plsc.addupdate_scatter(acc_x, [b], -ex, mask=m)
            plsc.addupdate_scatter(acc_y, [b], -ey, mask=m)
            plsc.addupdate_scatter(acc_z, [b], -ez, mask=m)
            return (cxs + _splat_last(lx), cys + _splat_last(ly),
                    czs + _splat_last(lz))

        def group_body(j, carry):
            a = j * (4 * L)
            for u in range(4):
                carry = one_group(a + u * L, carry)
            return carry

        carry = lax.fori_loop(0, G // 4, group_body, carry)

    last_boff = ((NCHUNK - 1) % 2) * BSTRIDE
    vlast = b_buf[pl.ds(last_boff + C, L)]
    cxs, cys, czs = carry
    plsc.addupdate_scatter(acc_x, [vlast], cxs, mask=mask15)
    plsc.addupdate_scatter(acc_y, [vlast], cys, mask=mask15)
    plsc.addupdate_scatter(acc_z, [vlast], czs, mask=mask15)

    row = sid * (3 * S)
    pltpu.sync_copy(acc_x, shared.at[pl.ds(row, S)])
    pltpu.sync_copy(acc_y, shared.at[pl.ds(row + S, S)])
    pltpu.sync_copy(acc_z, shared.at[pl.ds(row + 2 * S, S)])
    plsc.subcore_barrier()

    for t in range(NSUB):
        pltpu.sync_copy(shared.at[pl.ds(t * 3 * S + sid * W, W)],
                        red.at[pl.ds(t * W, W)])

    def red_body(j, _):
        v = red[pl.ds(j * L, L)]
        for t in range(1, NSUB):
            v = v + red[pl.ds(t * W + j * L, L)]
        out_loc[pl.ds(j * L, L)] = v
        return 0
    lax.fori_loop(0, W // L, red_body, 0)

    pltpu.sync_copy(out_loc, out_hbm.at[pl.ds(cid * 3 * S + sid * W, W)])


def _tc_combine(x_ref, o_ref):
    o_ref[...] = x_ref[0] + x_ref[1]


def kernel(positions, q, batch):
    xp = positions[:, 0]
    yp = positions[:, 1]
    zp = positions[:, 2]
    qf = q.reshape(-1)
    b = batch.astype(jnp.int32)

    mesh = plsc.VectorSubcoreMesh(core_axis_name="c", subcore_axis_name="s",
                                  num_cores=NCORES, num_subcores=NSUB)
    sc = pl.kernel(
        _sc_body,
        out_type=jax.ShapeDtypeStruct((NCORES * 3 * S,), jnp.float32),
        mesh=mesh,
        compiler_params=pltpu.CompilerParams(needs_layout_passes=False),
        scratch_types=[
            pltpu.VMEM((2 * C,), jnp.float32),
            pltpu.VMEM((2 * C,), jnp.float32),
            pltpu.VMEM((2 * C,), jnp.float32),
            pltpu.VMEM((2 * C,), jnp.float32),
            pltpu.VMEM((2 * BSTRIDE,), jnp.int32),
            pltpu.VMEM((S,), jnp.float32),
            pltpu.VMEM((S,), jnp.float32),
            pltpu.VMEM((S,), jnp.float32),
            pltpu.VMEM_SHARED((NSUB * 3 * S,), jnp.float32),
            pltpu.VMEM((NSUB * W,), jnp.float32),
            pltpu.VMEM((W,), jnp.float32),
            pltpu.SemaphoreType.DMA,
            pltpu.SemaphoreType.DMA,
            pltpu.SemaphoreType.DMA,
            pltpu.SemaphoreType.DMA,
            pltpu.SemaphoreType.DMA,
            pltpu.SemaphoreType.DMA,
            pltpu.SemaphoreType.DMA,
            pltpu.SemaphoreType.DMA,
            pltpu.SemaphoreType.DMA,
            pltpu.SemaphoreType.DMA,
        ],
    )
    partials = sc(xp, yp, zp, qf, b)

    planar = pl.pallas_call(
        _tc_combine,
        out_shape=jax.ShapeDtypeStruct((3, S), jnp.float32),
    )(partials.reshape(NCORES, 3, S))
    return planar.T
